# baseline (speedup 1.0000x reference)
import functools

import jax
import jax.numpy as jnp
from jax import lax
from jax.experimental import pallas as pl
from jax.experimental.pallas import tpu as pltpu
from jax.experimental.pallas import tpu_sc as plsc

BATCH = 16384
D = 64
NC = 2
NS = 16
NW = NC * NS
BPW = BATCH // NW
CHUNK = 128
NCHUNK = BPW // CHUNK


def _bpr_body(user_h, pos_h, neg_h, ut_h, it_h, outp_h, outn_h,
              uidx, pidx, nidx, urows, prows, nrows, outp_v, outn_v, sem):
    wid = lax.axis_index("s") * NC + lax.axis_index("c")
    base = wid * BPW

    pltpu.sync_copy(user_h.at[pl.ds(base, BPW)], uidx)
    pltpu.sync_copy(pos_h.at[pl.ds(base, BPW)], pidx)
    pltpu.sync_copy(neg_h.at[pl.ds(base, BPW)], nidx)

    copies = []
    for j in range(NCHUNK):
        sl = pl.ds(j * CHUNK, CHUNK)
        copies.append(pltpu.async_copy(ut_h.at[uidx.at[sl]], urows.at[sl], sem))
        copies.append(pltpu.async_copy(it_h.at[pidx.at[sl]], prows.at[sl], sem))
        copies.append(pltpu.async_copy(it_h.at[nidx.at[sl]], nrows.at[sl], sem))
    for c in copies:
        c.wait()

    def group(g, carry):
        rows = g * 16 + lax.iota(jnp.int32, 16)

        def col(c, accs):
            ap, an = accs
            cols = jnp.full((16,), c, jnp.int32)
            u = plsc.load_gather(urows, [rows, cols])
            p = plsc.load_gather(prows, [rows, cols])
            n = plsc.load_gather(nrows, [rows, cols])
            return ap + u * p, an + u * n

        zero = jnp.zeros((16,), jnp.float32)
        ap, an = lax.fori_loop(0, D, col, (zero, zero))
        outp_v[pl.ds(g * 16, 16)] = ap
        outn_v[pl.ds(g * 16, 16)] = an
        return carry

    lax.fori_loop(0, BPW // 16, group, 0)

    pltpu.sync_copy(outp_v, outp_h.at[pl.ds(base, BPW)])
    pltpu.sync_copy(outn_v, outn_h.at[pl.ds(base, BPW)])


@jax.jit
def kernel(user, pos_item, neg_item, user_table, item_table):
    mesh = plsc.VectorSubcoreMesh(core_axis_name="c", subcore_axis_name="s")
    f = pl.kernel(
        _bpr_body,
        mesh=mesh,
        compiler_params=pltpu.CompilerParams(
            needs_layout_passes=False, use_tc_tiling_on_sc=False),
        out_type=(
            jax.ShapeDtypeStruct((BATCH,), jnp.float32),
            jax.ShapeDtypeStruct((BATCH,), jnp.float32),
        ),
        scratch_types=[
            pltpu.VMEM((BPW,), jnp.int32),
            pltpu.VMEM((BPW,), jnp.int32),
            pltpu.VMEM((BPW,), jnp.int32),
            pltpu.VMEM((BPW, D), jnp.float32),
            pltpu.VMEM((BPW, D), jnp.float32),
            pltpu.VMEM((BPW, D), jnp.float32),
            pltpu.VMEM((BPW,), jnp.float32),
            pltpu.VMEM((BPW,), jnp.float32),
            pltpu.SemaphoreType.DMA,
        ],
    )
    return f(user, pos_item, neg_item, user_table, item_table)
